# baseline (speedup 1.0000x reference)
import functools

import jax
import jax.numpy as jnp
from jax import lax
from jax.experimental import pallas as pl
from jax.experimental.pallas import tpu as pltpu
from jax.experimental.pallas import tpu_sc as plsc

_B, _L, _V, _E, _A = 1024, 200, 100000, 128, 8
_SPLIT = 128
_REM = _L - _SPLIT
_OFFS = tuple(range(0, _L - 16, 16)) + (_L - 16,)


def kernel(sequence, attrs_idxs, token_table, pos_table, attr_table0,
           attr_table1):
    seq_flat = sequence.reshape(_B * _L)
    a0 = attrs_idxs[0].reshape(_B * _L)
    a1 = attrs_idxs[1].reshape(_B * _L)

    info = plsc.get_sparse_core_info()
    nc, ns = info.num_cores, info.num_subcores
    nw = nc * ns
    rows_per_w = _B // nw

    mesh = plsc.VectorSubcoreMesh(core_axis_name="c", subcore_axis_name="s")

    @functools.partial(
        pl.kernel,
        mesh=mesh,
        out_type=jax.ShapeDtypeStruct((_B, _L, _E), jnp.float32),
        scratch_types=[
            pltpu.VMEM((_L, _E), jnp.float32),
            pltpu.VMEM((_A, _E), jnp.float32),
            pltpu.VMEM((_A, _E), jnp.float32),
            pltpu.VMEM((_A * _A, _E), jnp.float32),
            pltpu.VMEM((_L,), jnp.int32),
            pltpu.VMEM((_L,), jnp.int32),
            pltpu.VMEM((_L,), jnp.int32),
            pltpu.VMEM((_L,), jnp.int32),
            pltpu.VMEM((_L,), jnp.int32),
            pltpu.VMEM((_L,), jnp.int32),
            pltpu.VMEM((_L + 16,), jnp.int32),
            pltpu.VMEM((_L + 16,), jnp.int32),
            pltpu.VMEM((_L, _E), jnp.float32),
            pltpu.VMEM((_L, _E), jnp.float32),
            pltpu.SemaphoreType.DMA,
            pltpu.SemaphoreType.DMA,
            pltpu.SemaphoreType.DMA,
            pltpu.SemaphoreType.DMA,
        ],
    )
    def k(seq_hbm, a0_hbm, a1_hbm, token_hbm, pos_hbm, attr0_hbm, attr1_hbm,
          out_hbm, pos_v, attr0_v, attr1_v, pair_v,
          seqx0, seqx1, a0b0, a0b1, a1b0, a1b1, pidx0, pidx1,
          rows0, rows1, sem_g0, sem_g1, sem_w0, sem_w1):
        wid = lax.axis_index("s") * nc + lax.axis_index("c")
        b0 = wid * rows_per_w

        pltpu.sync_copy(pos_hbm, pos_v)
        pltpu.sync_copy(attr0_hbm, attr0_v)
        pltpu.sync_copy(attr1_hbm, attr1_v)

        def build_pair(i, carry):
            for j in range(_A):
                for cb in range(_E // 16):
                    s = pl.ds(cb * 16, 16)
                    pair_v[i * _A + j, s] = attr0_v[i, s] + attr1_v[j, s]
            return carry

        lax.fori_loop(0, _A, build_pair, 0)

        bufs = (
            (seqx0, a0b0, a1b0, pidx0, rows0, sem_g0, sem_w0),
            (seqx1, a0b1, a1b1, pidx1, rows1, sem_g1, sem_w1),
        )

        def stage_idx(c, buf):
            seqx, a0b, a1b, pidx = buf[0], buf[1], buf[2], buf[3]
            base = (b0 + c) * _L
            pltpu.sync_copy(seq_hbm.at[pl.ds(base, _SPLIT)],
                            seqx.at[pl.ds(0, _SPLIT)])
            pltpu.sync_copy(seq_hbm.at[pl.ds(base + _SPLIT, _REM)],
                            seqx.at[pl.ds(_SPLIT, _REM)])
            pltpu.sync_copy(a0_hbm.at[pl.ds(base, _L)], a0b)
            pltpu.sync_copy(a1_hbm.at[pl.ds(base, _L)], a1b)
            for off in _OFFS:
                s = pl.ds(off, 16)
                pidx[s] = a0b[s] * _A + a1b[s]

        def gather_copies(buf):
            seqx, rows, sem_g = buf[0], buf[4], buf[5]
            cp1 = pltpu.make_async_copy(
                token_hbm.at[seqx.at[pl.ds(0, _SPLIT)]],
                rows.at[pl.ds(0, _SPLIT)], sem_g)
            cp2 = pltpu.make_async_copy(
                token_hbm.at[seqx.at[pl.ds(_SPLIT, _REM)]],
                rows.at[pl.ds(_SPLIT, _REM)], sem_g)
            return cp1, cp2

        def step(c, cur, nxt):
            b = b0 + c

            @pl.when(c < rows_per_w - 1)
            def _():
                stage_idx(c + 1, nxt)

            @pl.when(c > 0)
            def _():
                pltpu.make_async_copy(nxt[4], out_hbm.at[b - 1],
                                      nxt[6]).wait()

            @pl.when(c < rows_per_w - 1)
            def _():
                for cp in gather_copies(nxt):
                    cp.start()

            for cp in gather_copies(cur):
                cp.wait()

            pidx, rows = cur[3], cur[4]

            def row(r, rcarry):
                pv = pidx[pl.ds(r, 16)]
                p = pv[0]
                for cb in range(_E // 16):
                    s = pl.ds(cb * 16, 16)
                    rows[r, s] = rows[r, s] + pos_v[r, s] + pair_v[p, s]
                return rcarry

            lax.fori_loop(0, _L, row, 0, unroll=2)
            pltpu.make_async_copy(rows, out_hbm.at[b], cur[6]).start()

        stage_idx(0, bufs[0])
        for cp in gather_copies(bufs[0]):
            cp.start()

        def pair_of_chunks(i, carry):
            step(2 * i, bufs[0], bufs[1])
            step(2 * i + 1, bufs[1], bufs[0])
            return carry

        lax.fori_loop(0, rows_per_w // 2, pair_of_chunks, 0)
        pltpu.make_async_copy(bufs[1][4], out_hbm.at[b0 + rows_per_w - 1],
                              bufs[1][6]).wait()

    return k(seq_flat, a0, a1, token_table, pos_table, attr_table0,
             attr_table1)


# speedup vs baseline: 1.1178x; 1.1178x over previous
import functools

import jax
import jax.numpy as jnp
from jax import lax
from jax.experimental import pallas as pl
from jax.experimental.pallas import tpu as pltpu
from jax.experimental.pallas import tpu_sc as plsc

_B, _L, _V, _E, _A = 1024, 200, 100000, 128, 8
_SPLIT = 128
_REM = _L - _SPLIT
_OFFS = tuple(range(0, _L - 16, 16)) + (_L - 16,)


def kernel(sequence, attrs_idxs, token_table, pos_table, attr_table0,
           attr_table1):
    seq_flat = sequence.reshape(_B * _L)
    a0 = attrs_idxs[0].reshape(_B * _L)
    a1 = attrs_idxs[1].reshape(_B * _L)

    info = plsc.get_sparse_core_info()
    nc, ns = info.num_cores, info.num_subcores
    nw = nc * ns
    rows_per_w = _B // nw

    mesh = plsc.VectorSubcoreMesh(core_axis_name="c", subcore_axis_name="s")

    @functools.partial(
        pl.kernel,
        mesh=mesh,
        out_type=jax.ShapeDtypeStruct((_B, _L, _E), jnp.float32),
        scratch_types=[
            pltpu.VMEM((_L, _E), jnp.float32),
            pltpu.VMEM((_A, _E), jnp.float32),
            pltpu.VMEM((_A, _E), jnp.float32),
            pltpu.VMEM((_A * _A, _E), jnp.float32),
            pltpu.VMEM((_L,), jnp.int32),
            pltpu.VMEM((_L,), jnp.int32),
            pltpu.VMEM((_L,), jnp.int32),
            pltpu.VMEM((_L,), jnp.int32),
            pltpu.VMEM((_L,), jnp.int32),
            pltpu.VMEM((_L,), jnp.int32),
            pltpu.VMEM((_L + 16,), jnp.int32),
            pltpu.VMEM((_L + 16,), jnp.int32),
            pltpu.VMEM((_L, _E), jnp.float32),
            pltpu.VMEM((_L, _E), jnp.float32),
            pltpu.SemaphoreType.DMA,
            pltpu.SemaphoreType.DMA,
            pltpu.SemaphoreType.DMA,
            pltpu.SemaphoreType.DMA,
        ],
    )
    def k(seq_hbm, a0_hbm, a1_hbm, token_hbm, pos_hbm, attr0_hbm, attr1_hbm,
          out_hbm, pos_v, attr0_v, attr1_v, pair_v,
          seqx0, seqx1, a0b0, a0b1, a1b0, a1b1, pidx0, pidx1,
          rows0, rows1, sem_g0, sem_g1, sem_w0, sem_w1):
        wid = lax.axis_index("s") * nc + lax.axis_index("c")
        b0 = wid * rows_per_w

        pltpu.sync_copy(pos_hbm, pos_v)
        pltpu.sync_copy(attr0_hbm, attr0_v)
        pltpu.sync_copy(attr1_hbm, attr1_v)

        def build_pair(i, carry):
            for j in range(_A):
                for cb in range(_E // 16):
                    s = pl.ds(cb * 16, 16)
                    pair_v[i * _A + j, s] = attr0_v[i, s] + attr1_v[j, s]
            return carry

        lax.fori_loop(0, _A, build_pair, 0)

        bufs = (
            (seqx0, a0b0, a1b0, pidx0, rows0, sem_g0, sem_w0),
            (seqx1, a0b1, a1b1, pidx1, rows1, sem_g1, sem_w1),
        )

        def stage_idx(c, buf):
            seqx, a0b, a1b, pidx = buf[0], buf[1], buf[2], buf[3]
            base = (b0 + c) * _L
            pltpu.sync_copy(seq_hbm.at[pl.ds(base, _SPLIT)],
                            seqx.at[pl.ds(0, _SPLIT)])
            pltpu.sync_copy(seq_hbm.at[pl.ds(base + _SPLIT, _REM)],
                            seqx.at[pl.ds(_SPLIT, _REM)])
            pltpu.sync_copy(a0_hbm.at[pl.ds(base, _L)], a0b)
            pltpu.sync_copy(a1_hbm.at[pl.ds(base, _L)], a1b)
            for off in _OFFS:
                s = pl.ds(off, 16)
                pidx[s] = a0b[s] * _A + a1b[s]

        def gather_copies(buf):
            seqx, rows, sem_g = buf[0], buf[4], buf[5]
            cp1 = pltpu.make_async_copy(
                token_hbm.at[seqx.at[pl.ds(0, _SPLIT)]],
                rows.at[pl.ds(0, _SPLIT)], sem_g)
            cp2 = pltpu.make_async_copy(
                token_hbm.at[seqx.at[pl.ds(_SPLIT, _REM)]],
                rows.at[pl.ds(_SPLIT, _REM)], sem_g)
            return cp1, cp2

        def step(c, cur, nxt):
            b = b0 + c

            @pl.when(c < rows_per_w - 1)
            def _():
                stage_idx(c + 1, nxt)

            @pl.when(c > 0)
            def _():
                pltpu.make_async_copy(nxt[4], out_hbm.at[b - 1],
                                      nxt[6]).wait()

            @pl.when(c < rows_per_w - 1)
            def _():
                for cp in gather_copies(nxt):
                    cp.start()

            for cp in gather_copies(cur):
                cp.wait()

            pidx, rows = cur[3], cur[4]

            def row(r, rcarry):
                pv = pidx[pl.ds(r, 16)]
                p = pv[0]
                for cb in range(_E // 16):
                    s = pl.ds(cb * 16, 16)
                    rows[r, s] = rows[r, s] + pos_v[r, s] + pair_v[p, s]
                return rcarry

            lax.fori_loop(0, _L, row, 0)
            pltpu.make_async_copy(rows, out_hbm.at[b], cur[6]).start()

        stage_idx(0, bufs[0])
        for cp in gather_copies(bufs[0]):
            cp.start()

        def pair_of_chunks(i, carry):
            step(2 * i, bufs[0], bufs[1])
            step(2 * i + 1, bufs[1], bufs[0])
            return carry

        lax.fori_loop(0, rows_per_w // 2, pair_of_chunks, 0)
        pltpu.make_async_copy(bufs[1][4], out_hbm.at[b0 + rows_per_w - 1],
                              bufs[1][6]).wait()

    return k(seq_flat, a0, a1, token_table, pos_table, attr_table0,
             attr_table1)
